# baseline (speedup 1.0000x reference)
import functools

import jax
import jax.numpy as jnp
from jax import lax
from jax.experimental import pallas as pl
from jax.experimental.pallas import tpu as pltpu
from jax.experimental.pallas import tpu_sc as plsc

_NC = 2
_NS = 16
_NW = _NC * _NS

_H = 200
_BLK = 128
_HC = 4
_CHUNK = _HC * _BLK
_B_PER_W = _H * _BLK
_N_CHUNKS = _B_PER_W // _CHUNK
_D = 32


def _sc_gather(table, idx_flat):
    mesh = plsc.VectorSubcoreMesh(core_axis_name="c", subcore_axis_name="s")

    @functools.partial(
        pl.kernel,
        mesh=mesh,
        out_type=jax.ShapeDtypeStruct((_H, 4, _NW, 8, _BLK), jnp.float32),
        scratch_types=[
            pltpu.VMEM((_B_PER_W,), jnp.int32),
            pltpu.VMEM((_B_PER_W,), jnp.int32),
            pltpu.VMEM((2, _CHUNK, _D), jnp.float32),
            pltpu.VMEM((2, _HC, 4, 8, _BLK), jnp.float32),
            pltpu.SemaphoreType.DMA((2,)),
            pltpu.SemaphoreType.DMA((2,)),
        ],
        compiler_params=pltpu.CompilerParams(
            use_tc_tiling_on_sc=False, needs_layout_passes=False
        ),
    )
    def k(table_hbm, idx_hbm, out_hbm, idx_v, ridx_v, rows_v, xbuf_v,
          gsem, osem):
        wid = lax.axis_index("s") * _NC + lax.axis_index("c")
        base = wid * _B_PER_W

        pltpu.sync_copy(idx_hbm.at[pl.ds(base, _B_PER_W)], idx_v)

        lanes = lax.iota(jnp.int32, 16)
        lanes_h = lanes * _H

        def reorder(h, carry):
            vs = [
                plsc.load_gather(idx_v, [lanes_h + (i * 16 * _H + h)])
                for i in range(_BLK // 16)
            ]
            for i, v in enumerate(vs):
                ridx_v[pl.ds(h * _BLK + i * 16, 16)] = v
            return carry

        lax.fori_loop(0, _H, reorder, 0)

        def start_gather(chunk, buf):
            return pltpu.async_copy(
                table_hbm.at[ridx_v.at[pl.ds(chunk * _CHUNK, _CHUNK)]],
                rows_v.at[buf],
                gsem.at[buf],
            )

        def wait_gather(buf):
            pltpu.make_async_copy(
                table_hbm.at[ridx_v.at[pl.ds(0, _CHUNK)]],
                rows_v.at[buf],
                gsem.at[buf],
            ).wait()

        def start_write(chunk, buf):
            h0 = chunk * _HC
            return pltpu.async_copy(
                xbuf_v.at[buf],
                out_hbm.at[pl.ds(h0, _HC), :, wid],
                osem.at[buf],
            )

        def wait_write(buf):
            pltpu.make_async_copy(
                xbuf_v.at[buf],
                out_hbm.at[pl.ds(0, _HC), :, wid],
                osem.at[buf],
            ).wait()

        dzero = lanes * 0

        def transpose(buf):
            rows = rows_v.at[buf]
            xb = xbuf_v.at[buf]

            def h_body(h_loc, carry):
                rbase = h_loc * _BLK
                for c0 in range(0, _BLK, 16):
                    rvec = lanes + (rbase + c0)
                    vs = [
                        plsc.load_gather(rows, [rvec, dzero + d])
                        for d in range(_D)
                    ]
                    for d, v in enumerate(vs):
                        xb[h_loc, d // 8, d % 8, pl.ds(c0, 16)] = v
                return carry

            lax.fori_loop(0, _HC, h_body, 0)

        start_gather(0, 0)

        def body(i, carry):
            wait_gather(0)
            start_gather(2 * i + 1, 1)

            @pl.when(i > 0)
            def _():
                wait_write(0)

            transpose(0)
            start_write(2 * i, 0)

            wait_gather(1)

            @pl.when(i < _N_CHUNKS // 2 - 1)
            def _():
                start_gather(2 * i + 2, 0)

            @pl.when(i > 0)
            def _():
                wait_write(1)

            transpose(1)
            start_write(2 * i + 1, 1)
            return carry

        lax.fori_loop(0, _N_CHUNKS // 2, body, 0)
        wait_write(0)
        wait_write(1)

    return k(table, idx_flat)


def _tc_repack(wt):
    def body(in_ref, out_ref):
        rowi = lax.broadcasted_iota(jnp.int32, (32, 32), 0)
        coli = lax.broadcasted_iota(jnp.int32, (32, 32), 1)
        eye = (rowi == coli).astype(jnp.float32)
        x = in_ref[...]
        xt = jax.lax.dot_general(
            x, eye, (((0,), (0,)), ((), ())),
            preferred_element_type=jnp.float32,
        )
        xt = xt.reshape(512, 4, 32)
        out_ref[...] = jnp.concatenate(
            [xt[:, u, :] for u in range(4)], axis=1
        )

    return pl.pallas_call(
        body,
        grid=(489,),
        in_specs=[pl.BlockSpec((32, 2048), lambda i: (0, i))],
        out_specs=pl.BlockSpec((512, 128), lambda i: (i, 0)),
        out_shape=jax.ShapeDtypeStruct((250000, 128), jnp.float32),
    )(wt)


def kernel(token_ids, weight):
    idx = token_ids.reshape(-1).astype(jnp.int32)
    w128 = _tc_repack(weight.T)
    x = _sc_gather(w128.reshape(1000000, 32), idx)
    out = x.transpose(2, 4, 0, 1, 3)
    return out.reshape(4096, 200, 32)


# speedup vs baseline: 1.0331x; 1.0331x over previous
import functools

import jax
import jax.numpy as jnp
from jax import lax
from jax.experimental import pallas as pl
from jax.experimental.pallas import tpu as pltpu
from jax.experimental.pallas import tpu_sc as plsc

_NC = 2
_NS = 16
_NW = _NC * _NS

_H = 200
_BLK = 128
_HC = 4
_CHUNK = _HC * _BLK
_B_PER_W = _H * _BLK
_N_CHUNKS = _B_PER_W // _CHUNK
_D = 32


def _sc_gather(table, idx_flat):
    mesh = plsc.VectorSubcoreMesh(core_axis_name="c", subcore_axis_name="s")

    @functools.partial(
        pl.kernel,
        mesh=mesh,
        out_type=jax.ShapeDtypeStruct((_H, 4, _NW, 8, _BLK), jnp.float32),
        scratch_types=[
            pltpu.VMEM((_B_PER_W,), jnp.int32),
            pltpu.VMEM((_B_PER_W,), jnp.int32),
            pltpu.VMEM((2, _CHUNK, _D), jnp.float32),
            pltpu.VMEM((2, _HC, 4, 8, _BLK), jnp.float32),
            pltpu.SemaphoreType.DMA((2,)),
            pltpu.SemaphoreType.DMA((2,)),
        ],
        compiler_params=pltpu.CompilerParams(
            use_tc_tiling_on_sc=False, needs_layout_passes=False
        ),
    )
    def k(table_hbm, idx_hbm, out_hbm, idx_v, ridx_v, rows_v, xbuf_v,
          gsem, osem):
        wid = lax.axis_index("s") * _NC + lax.axis_index("c")
        base = wid * _B_PER_W

        pltpu.sync_copy(idx_hbm.at[pl.ds(base, _B_PER_W)], idx_v)

        lanes = lax.iota(jnp.int32, 16)
        lanes_h = lanes * _H

        def reorder(h, carry):
            vs = [
                plsc.load_gather(idx_v, [lanes_h + (i * 16 * _H + h)])
                for i in range(_BLK // 16)
            ]
            for i, v in enumerate(vs):
                ridx_v[pl.ds(h * _BLK + i * 16, 16)] = v
            return carry

        lax.fori_loop(0, _H, reorder, 0)

        def start_gather(chunk, buf):
            return pltpu.async_copy(
                table_hbm.at[ridx_v.at[pl.ds(chunk * _CHUNK, _CHUNK)]],
                rows_v.at[buf],
                gsem.at[buf],
            )

        def wait_gather(buf):
            pltpu.make_async_copy(
                table_hbm.at[ridx_v.at[pl.ds(0, _CHUNK)]],
                rows_v.at[buf],
                gsem.at[buf],
            ).wait()

        def start_write(chunk, buf):
            h0 = chunk * _HC
            return pltpu.async_copy(
                xbuf_v.at[buf],
                out_hbm.at[pl.ds(h0, _HC), :, wid],
                osem.at[buf],
            )

        def wait_write(buf):
            pltpu.make_async_copy(
                xbuf_v.at[buf],
                out_hbm.at[pl.ds(0, _HC), :, wid],
                osem.at[buf],
            ).wait()

        dzero = lanes * 0

        def transpose(buf):
            rows = rows_v.at[buf]
            xb = xbuf_v.at[buf]

            def h_body(h_loc, carry):
                rbase = h_loc * _BLK
                for c0 in range(0, _BLK, 16):
                    rvec = lanes + (rbase + c0)
                    vs = [
                        plsc.load_gather(rows, [rvec, dzero + d])
                        for d in range(_D)
                    ]
                    for d, v in enumerate(vs):
                        xb[h_loc, d // 8, d % 8, pl.ds(c0, 16)] = v
                return carry

            lax.fori_loop(0, _HC, h_body, 0)

        start_gather(0, 0)

        def body(i, carry):
            wait_gather(0)
            start_gather(2 * i + 1, 1)

            @pl.when(i > 0)
            def _():
                wait_write(0)

            transpose(0)
            start_write(2 * i, 0)

            wait_gather(1)

            @pl.when(i < _N_CHUNKS // 2 - 1)
            def _():
                start_gather(2 * i + 2, 0)

            @pl.when(i > 0)
            def _():
                wait_write(1)

            transpose(1)
            start_write(2 * i + 1, 1)
            return carry

        lax.fori_loop(0, _N_CHUNKS // 2, body, 0)
        wait_write(0)
        wait_write(1)

    return k(table, idx_flat)


def _tc_repack(wt):
    def body(in_ref, out_ref):
        xt = in_ref[...].T.reshape(512, 4, 32)
        out_ref[...] = jnp.concatenate(
            [xt[:, u, :] for u in range(4)], axis=1
        )

    return pl.pallas_call(
        body,
        grid=(489,),
        in_specs=[pl.BlockSpec((32, 2048), lambda i: (0, i))],
        out_specs=pl.BlockSpec((512, 128), lambda i: (i, 0)),
        out_shape=jax.ShapeDtypeStruct((250000, 128), jnp.float32),
    )(wt)


def kernel(token_ids, weight):
    idx = token_ids.reshape(-1).astype(jnp.int32)
    w128 = _tc_repack(weight.T)
    x = _sc_gather(w128.reshape(1000000, 32), idx)
    out = x.transpose(2, 4, 0, 1, 3)
    return out.reshape(4096, 200, 32)
